# Initial kernel scaffold; baseline (speedup 1.0000x reference)
import jax
import jax.numpy as jnp
from jax.experimental import pallas as pl


def kernel(x, idx, x_skip, pos_skip, batch_skip, point2curveidx_skip, W, b):
    raise NotImplementedError("write your pallas kernel here")


# speedup vs baseline: 9.7974x; 9.7974x over previous
import functools

import jax
import jax.numpy as jnp
from jax import lax
from jax.experimental import pallas as pl
from jax.experimental.pallas import tpu as pltpu
from jax.experimental.pallas import tpu_sc as plsc

N_FINE = 16384
N_COARSE = 4096
D = 256
K = 3
BIG = 1e10
MASKED = 1e30

NC = 2
NS = 16
NW = NC * NS
L = 16

RF = 256
NBLK = N_FINE // RF
YBLK = N_COARSE // RF

CPW = N_COARSE // NW
CB = 32
PTS_W = N_FINE // NW
NCHUNK = PTS_W // CB

@functools.cache
def _sc_kernels():
    mesh = plsc.VectorSubcoreMesh(
        core_axis_name="c", subcore_axis_name="s",
        num_cores=NC, num_subcores=NS)

    @functools.partial(
        pl.kernel,
        out_type=[jax.ShapeDtypeStruct((N_COARSE,), jnp.float32)] * 4,
        mesh=mesh,
        scratch_types=[
            pltpu.VMEM((N_FINE,), jnp.float32),
            pltpu.VMEM((N_FINE,), jnp.float32),
            pltpu.VMEM((N_FINE,), jnp.float32),
            pltpu.VMEM((N_FINE,), jnp.float32),
            pltpu.VMEM((CPW,), jnp.int32),
            pltpu.VMEM((CPW,), jnp.float32),
            pltpu.VMEM((CPW,), jnp.float32),
            pltpu.VMEM((CPW,), jnp.float32),
            pltpu.VMEM((CPW,), jnp.float32),
        ],
        compiler_params=pltpu.CompilerParams(needs_layout_passes=False),
    )
    def _sc_gather_coarse(px, py, pz, pb, idxh, opx, opy, opz, opb,
                          tx, ty, tz, tb, idx_v, ox, oy, oz, ob):
        wid = lax.axis_index("s") * NC + lax.axis_index("c")
        base = wid * CPW
        pltpu.sync_copy(px, tx)
        pltpu.sync_copy(py, ty)
        pltpu.sync_copy(pz, tz)
        pltpu.sync_copy(pb, tb)
        pltpu.sync_copy(idxh.at[pl.ds(base, CPW)], idx_v)

        def body(j, _):
            sl = pl.ds(j * L, L)
            iv = idx_v[sl]
            ox[sl] = plsc.load_gather(tx, [iv])
            oy[sl] = plsc.load_gather(ty, [iv])
            oz[sl] = plsc.load_gather(tz, [iv])
            ob[sl] = plsc.load_gather(tb, [iv])
            return 0

        lax.fori_loop(0, CPW // L, body, 0)
        pltpu.sync_copy(ox, opx.at[pl.ds(base, CPW)])
        pltpu.sync_copy(oy, opy.at[pl.ds(base, CPW)])
        pltpu.sync_copy(oz, opz.at[pl.ds(base, CPW)])
        pltpu.sync_copy(ob, opb.at[pl.ds(base, CPW)])

    @functools.partial(
        pl.kernel,
        out_type=jax.ShapeDtypeStruct((N_FINE, D), jnp.float32),
        mesh=mesh,
        scratch_types=[
            pltpu.VMEM((3 * CB,), jnp.int32),
            pltpu.VMEM((3 * CB,), jnp.float32),
            pltpu.VMEM((3 * CB, D), jnp.float32),
            pltpu.VMEM((CB, D), jnp.float32),
            pltpu.VMEM((CB, D), jnp.float32),
            pltpu.SemaphoreType.DMA,
        ],
        compiler_params=pltpu.CompilerParams(needs_layout_passes=False),
    )
    def _sc_combine(y_hbm, z_hbm, nn_hbm, w_hbm, out_hbm,
                    idx_v, w_v, g_v, z_v, o_v, sem):
        wid = lax.axis_index("s") * NC + lax.axis_index("c")

        def chunk_body(c, _):
            pbase = wid * PTS_W + c * CB
            fbase = pbase * 3
            pltpu.sync_copy(nn_hbm.at[pl.ds(fbase, 3 * CB)], idx_v)
            pltpu.sync_copy(w_hbm.at[pl.ds(fbase, 3 * CB)], w_v)
            pltpu.sync_copy(z_hbm.at[pl.ds(pbase, CB)], z_v)
            pltpu.async_copy(y_hbm.at[idx_v], g_v, sem).wait()

            def point_body(i, _):
                j0 = 3 * i
                w0 = plsc.load_gather(w_v, [jnp.full((L,), j0, jnp.int32)])
                w1 = plsc.load_gather(w_v, [jnp.full((L,), j0 + 1, jnp.int32)])
                w2 = plsc.load_gather(w_v, [jnp.full((L,), j0 + 2, jnp.int32)])
                for v in range(D // L):
                    sl = pl.ds(v * L, L)
                    o_v[i, sl] = (z_v[i, sl] + w0 * g_v[j0, sl]
                                  + w1 * g_v[j0 + 1, sl] + w2 * g_v[j0 + 2, sl])
                return 0

            lax.fori_loop(0, CB, point_body, 0)
            pltpu.sync_copy(o_v, out_hbm.at[pl.ds(pbase, CB)])
            return 0

        lax.fori_loop(0, NCHUNK, chunk_body, 0)

    return _sc_gather_coarse, _sc_combine


def _knn_mlp_body(posf_ref, batf_ref, posct_ref, batct_ref, xs_ref, x_ref,
                  w_ref, b_ref, z_ref, y_ref, wk_ref, ak_ref):
    i = pl.program_id(0)
    pf = posf_ref[...]
    d2 = None
    for d in range(3):
        diff = pf[:, d:d + 1] - posct_ref[d:d + 1, :]
        d2 = diff * diff if d2 is None else d2 + diff * diff
    cross = batf_ref[...] != batct_ref[...]
    d2 = jnp.where(cross, jnp.float32(BIG), d2)

    iota = lax.broadcasted_iota(jnp.int32, (RF, N_COARSE), 1)
    ms, asel = [], []
    cur = d2
    for k in range(K):
        m = jnp.min(cur, axis=1, keepdims=True)
        a = jnp.min(jnp.where(cur == m, iota, N_COARSE), axis=1, keepdims=True)
        ms.append(m)
        asel.append(a)
        if k < K - 1:
            cur = jnp.where(iota == a, jnp.float32(MASKED), cur)

    r = [1.0 / (m + jnp.float32(1e-8)) for m in ms]
    s = (r[0] + r[1]) + r[2] + jnp.float32(1e-16)
    wk_ref[...] = jnp.concatenate([ri / s for ri in r], axis=1)
    ak_ref[...] = jnp.concatenate(asel, axis=1)

    z_ref[...] = (jnp.dot(xs_ref[...], w_ref[D:, :],
                          preferred_element_type=jnp.float32) + b_ref[...])

    @pl.when(i < YBLK)
    def _():
        y_ref[...] = jnp.dot(x_ref[...], w_ref[:D, :],
                             preferred_element_type=jnp.float32)


def _tc_knn_mlp(pos_f, bat_f2, posct, batct, x_skip, x, W, b2):
    return pl.pallas_call(
        _knn_mlp_body,
        grid=(NBLK,),
        in_specs=[
            pl.BlockSpec((RF, 3), lambda i: (i, 0)),
            pl.BlockSpec((RF, 1), lambda i: (i, 0)),
            pl.BlockSpec((3, N_COARSE), lambda i: (0, 0)),
            pl.BlockSpec((1, N_COARSE), lambda i: (0, 0)),
            pl.BlockSpec((RF, D), lambda i: (i, 0)),
            pl.BlockSpec((RF, D), lambda i: (jnp.minimum(i, YBLK - 1), 0)),
            pl.BlockSpec((2 * D, D), lambda i: (0, 0)),
            pl.BlockSpec((1, D), lambda i: (0, 0)),
        ],
        out_specs=[
            pl.BlockSpec((RF, D), lambda i: (i, 0)),
            pl.BlockSpec((RF, D), lambda i: (jnp.minimum(i, YBLK - 1), 0)),
            pl.BlockSpec((RF, K), lambda i: (i, 0)),
            pl.BlockSpec((RF, K), lambda i: (i, 0)),
        ],
        out_shape=[
            jax.ShapeDtypeStruct((N_FINE, D), jnp.float32),
            jax.ShapeDtypeStruct((N_COARSE, D), jnp.float32),
            jax.ShapeDtypeStruct((N_FINE, K), jnp.float32),
            jax.ShapeDtypeStruct((N_FINE, K), jnp.int32),
        ],
        compiler_params=pltpu.CompilerParams(
            dimension_semantics=("arbitrary",)),
    )(pos_f, bat_f2, posct, batct, x_skip, x, W, b2)


def kernel(x, idx, x_skip, pos_skip, batch_skip, point2curveidx_skip, W, b):
    _sc_gather_coarse, _sc_combine = _sc_kernels()
    idx32 = idx.astype(jnp.int32)
    batf = batch_skip.astype(jnp.float32)
    px = pos_skip[:, 0]
    py = pos_skip[:, 1]
    pz = pos_skip[:, 2]

    pcx, pcy, pcz, pcb = _sc_gather_coarse(px, py, pz, batf, idx32)
    posct = jnp.stack([pcx, pcy, pcz], axis=0)
    batct = pcb.reshape(1, N_COARSE)

    z, y, wk, ak = _tc_knn_mlp(pos_skip, batf.reshape(N_FINE, 1), posct,
                               batct, x_skip, x, W, b.reshape(1, D))

    out = _sc_combine(y, z, ak.reshape(-1), wk.reshape(-1))
    return (out, pos_skip, batch_skip, point2curveidx_skip)


# speedup vs baseline: 13.7932x; 1.4078x over previous
import functools

import jax
import jax.numpy as jnp
from jax import lax
from jax.experimental import pallas as pl
from jax.experimental.pallas import tpu as pltpu
from jax.experimental.pallas import tpu_sc as plsc

N_FINE = 16384
N_COARSE = 4096
D = 256
K = 3
BIG = 1e10
MASKED = 1e30

NC = 2
NS = 16
NW = NC * NS
L = 16

RF = 256
NBLK = N_FINE // RF
YBLK = N_COARSE // RF

CPW = N_COARSE // NW
CB = 32
PTS_W = N_FINE // NW
NCHUNK = PTS_W // CB

@functools.cache
def _sc_kernels():
    mesh = plsc.VectorSubcoreMesh(
        core_axis_name="c", subcore_axis_name="s",
        num_cores=NC, num_subcores=NS)

    @functools.partial(
        pl.kernel,
        out_type=[jax.ShapeDtypeStruct((N_COARSE,), jnp.float32)] * 4,
        mesh=mesh,
        scratch_types=[
            pltpu.VMEM((N_FINE,), jnp.float32),
            pltpu.VMEM((N_FINE,), jnp.float32),
            pltpu.VMEM((N_FINE,), jnp.float32),
            pltpu.VMEM((N_FINE,), jnp.float32),
            pltpu.VMEM((CPW,), jnp.int32),
            pltpu.VMEM((CPW,), jnp.float32),
            pltpu.VMEM((CPW,), jnp.float32),
            pltpu.VMEM((CPW,), jnp.float32),
            pltpu.VMEM((CPW,), jnp.float32),
        ],
        compiler_params=pltpu.CompilerParams(needs_layout_passes=False),
    )
    def _sc_gather_coarse(px, py, pz, pb, idxh, opx, opy, opz, opb,
                          tx, ty, tz, tb, idx_v, ox, oy, oz, ob):
        wid = lax.axis_index("s") * NC + lax.axis_index("c")
        base = wid * CPW
        pltpu.sync_copy(px, tx)
        pltpu.sync_copy(py, ty)
        pltpu.sync_copy(pz, tz)
        pltpu.sync_copy(pb, tb)
        pltpu.sync_copy(idxh.at[pl.ds(base, CPW)], idx_v)

        def body(j, _):
            sl = pl.ds(j * L, L)
            iv = idx_v[sl]
            ox[sl] = plsc.load_gather(tx, [iv])
            oy[sl] = plsc.load_gather(ty, [iv])
            oz[sl] = plsc.load_gather(tz, [iv])
            ob[sl] = plsc.load_gather(tb, [iv])
            return 0

        lax.fori_loop(0, CPW // L, body, 0)
        pltpu.sync_copy(ox, opx.at[pl.ds(base, CPW)])
        pltpu.sync_copy(oy, opy.at[pl.ds(base, CPW)])
        pltpu.sync_copy(oz, opz.at[pl.ds(base, CPW)])
        pltpu.sync_copy(ob, opb.at[pl.ds(base, CPW)])

    @functools.partial(
        pl.kernel,
        out_type=jax.ShapeDtypeStruct((N_FINE, D), jnp.float32),
        mesh=mesh,
        scratch_types=[
            pltpu.VMEM((3 * CB,), jnp.int32),
            pltpu.VMEM((3 * CB,), jnp.float32),
            pltpu.VMEM((3 * CB, D), jnp.float32),
            pltpu.VMEM((CB, D), jnp.float32),
            pltpu.VMEM((CB, D), jnp.float32),
            pltpu.SemaphoreType.DMA,
        ],
        compiler_params=pltpu.CompilerParams(needs_layout_passes=False),
    )
    def _sc_combine(y_hbm, z_hbm, nn_hbm, w_hbm, out_hbm,
                    idx_v, w_v, g_v, z_v, o_v, sem):
        wid = lax.axis_index("s") * NC + lax.axis_index("c")

        def chunk_body(c, _):
            pbase = wid * PTS_W + c * CB
            fbase = pbase * 3
            pltpu.sync_copy(nn_hbm.at[pl.ds(fbase, 3 * CB)], idx_v)
            pltpu.sync_copy(w_hbm.at[pl.ds(fbase, 3 * CB)], w_v)
            pltpu.sync_copy(z_hbm.at[pl.ds(pbase, CB)], z_v)
            pltpu.async_copy(y_hbm.at[idx_v], g_v, sem).wait()

            def point_body(i, _):
                j0 = 3 * i
                w0 = plsc.load_gather(w_v, [jnp.full((L,), j0, jnp.int32)])
                w1 = plsc.load_gather(w_v, [jnp.full((L,), j0 + 1, jnp.int32)])
                w2 = plsc.load_gather(w_v, [jnp.full((L,), j0 + 2, jnp.int32)])
                for v in range(D // L):
                    sl = pl.ds(v * L, L)
                    o_v[i, sl] = (z_v[i, sl] + w0 * g_v[j0, sl]
                                  + w1 * g_v[j0 + 1, sl] + w2 * g_v[j0 + 2, sl])
                return 0

            lax.fori_loop(0, CB, point_body, 0)
            pltpu.sync_copy(o_v, out_hbm.at[pl.ds(pbase, CB)])
            return 0

        lax.fori_loop(0, NCHUNK, chunk_body, 0)

    return _sc_gather_coarse, _sc_combine


CW = 512
NCH = N_COARSE // CW


def _insert(m1, m2, m3, a1, a2, a3, v, iv):
    c1 = v < m1
    c2 = v < m2
    c3 = v < m3
    nm1 = jnp.where(c1, v, m1)
    na1 = jnp.where(c1, iv, a1)
    nm2 = jnp.where(c1, m1, jnp.where(c2, v, m2))
    na2 = jnp.where(c1, a1, jnp.where(c2, iv, a2))
    nm3 = jnp.where(c2, m2, jnp.where(c3, v, m3))
    na3 = jnp.where(c2, a2, jnp.where(c3, iv, a3))
    return nm1, nm2, nm3, na1, na2, na3


def _knn_mlp_body(posf_ref, batf_ref, posct3_ref, xs_ref, x_ref,
                  w_ref, b_ref, z_ref, y_ref, wk_ref, ak_ref):
    i = pl.program_id(0)
    pf = posf_ref[...]
    bf = batf_ref[...]
    b_min = batf_ref[0, 0]
    b_max = batf_ref[RF - 1, 0]
    bc_all = posct3_ref[3, :, :]
    lt = (bc_all < b_min).astype(jnp.int32)
    le = (bc_all <= b_max).astype(jnp.int32)
    c_lo = jnp.sum(jnp.sum(lt, axis=1, keepdims=True), axis=0,
                   keepdims=True)[0, 0]
    c_hi = jnp.sum(jnp.sum(le, axis=1, keepdims=True), axis=0,
                   keepdims=True)[0, 0]
    ch0 = c_lo // CW
    nch = (c_hi - ch0 * CW + (CW - 1)) // CW

    fiota_l = lax.broadcasted_iota(jnp.int32, (RF, CW), 1).astype(jnp.float32)

    def chunk_body(t, carry):
        m1, m2, m3, a1, a2, a3 = carry
        ch = ch0 + t
        d2 = None
        for d in range(3):
            pcd = posct3_ref[d, pl.ds(ch, 1), :]
            diff = pf[:, d:d + 1] - pcd
            d2 = diff * diff if d2 is None else d2 + diff * diff
        bc = posct3_ref[3, pl.ds(ch, 1), :]
        d2 = jnp.where(bf != bc, jnp.float32(BIG), d2)
        fiota = fiota_l + (ch * CW).astype(jnp.float32)
        cur = d2
        for k in range(K):
            m = jnp.min(cur, axis=1, keepdims=True)
            a = jnp.min(jnp.where(cur == m, fiota, jnp.float32(2 * N_COARSE)),
                        axis=1, keepdims=True)
            if k < K - 1:
                cur = jnp.where(fiota == a, jnp.float32(MASKED), cur)
            m1, m2, m3, a1, a2, a3 = _insert(m1, m2, m3, a1, a2, a3, m, a)
        return m1, m2, m3, a1, a2, a3

    init = (jnp.full((RF, 1), MASKED, jnp.float32),
            jnp.full((RF, 1), MASKED, jnp.float32),
            jnp.full((RF, 1), MASKED, jnp.float32),
            jnp.zeros((RF, 1), jnp.float32),
            jnp.ones((RF, 1), jnp.float32),
            jnp.full((RF, 1), 2.0, jnp.float32))
    m1, m2, m3, a1, a2, a3 = lax.fori_loop(0, nch, chunk_body, init)

    ms = [jnp.minimum(m, jnp.float32(BIG)) for m in (m1, m2, m3)]
    r = [1.0 / (m + jnp.float32(1e-8)) for m in ms]
    s = (r[0] + r[1]) + r[2] + jnp.float32(1e-16)
    wk_ref[...] = jnp.concatenate([ri / s for ri in r], axis=1)
    ak_ref[...] = jnp.concatenate(
        [a.astype(jnp.int32) for a in (a1, a2, a3)], axis=1)

    z_ref[...] = (jnp.dot(xs_ref[...], w_ref[D:, :],
                          preferred_element_type=jnp.float32) + b_ref[...])

    @pl.when(i < YBLK)
    def _():
        y_ref[...] = jnp.dot(x_ref[...], w_ref[:D, :],
                             preferred_element_type=jnp.float32)


def _tc_knn_mlp(pos_f, bat_f2, posct3, x_skip, x, W, b2):
    return pl.pallas_call(
        _knn_mlp_body,
        grid=(NBLK,),
        in_specs=[
            pl.BlockSpec((RF, 3), lambda i: (i, 0)),
            pl.BlockSpec((RF, 1), lambda i: (i, 0)),
            pl.BlockSpec((4, NCH, CW), lambda i: (0, 0, 0)),
            pl.BlockSpec((RF, D), lambda i: (i, 0)),
            pl.BlockSpec((RF, D), lambda i: (jnp.minimum(i, YBLK - 1), 0)),
            pl.BlockSpec((2 * D, D), lambda i: (0, 0)),
            pl.BlockSpec((1, D), lambda i: (0, 0)),
        ],
        out_specs=[
            pl.BlockSpec((RF, D), lambda i: (i, 0)),
            pl.BlockSpec((RF, D), lambda i: (jnp.minimum(i, YBLK - 1), 0)),
            pl.BlockSpec((RF, K), lambda i: (i, 0)),
            pl.BlockSpec((RF, K), lambda i: (i, 0)),
        ],
        out_shape=[
            jax.ShapeDtypeStruct((N_FINE, D), jnp.float32),
            jax.ShapeDtypeStruct((N_COARSE, D), jnp.float32),
            jax.ShapeDtypeStruct((N_FINE, K), jnp.float32),
            jax.ShapeDtypeStruct((N_FINE, K), jnp.int32),
        ],
        compiler_params=pltpu.CompilerParams(
            dimension_semantics=("arbitrary",)),
    )(pos_f, bat_f2, posct3, x_skip, x, W, b2)


def kernel(x, idx, x_skip, pos_skip, batch_skip, point2curveidx_skip, W, b):
    _sc_gather_coarse, _sc_combine = _sc_kernels()
    idx32 = idx.astype(jnp.int32)
    batf = batch_skip.astype(jnp.float32)
    px = pos_skip[:, 0]
    py = pos_skip[:, 1]
    pz = pos_skip[:, 2]

    pcx, pcy, pcz, pcb = _sc_gather_coarse(px, py, pz, batf, idx32)
    posct3 = jnp.stack([pcx, pcy, pcz, pcb], axis=0).reshape(4, NCH, CW)

    z, y, wk, ak = _tc_knn_mlp(pos_skip, batf.reshape(N_FINE, 1), posct3,
                               x_skip, x, W, b.reshape(1, D))

    out = _sc_combine(y, z, ak.reshape(-1), wk.reshape(-1))
    return (out, pos_skip, batch_skip, point2curveidx_skip)


# speedup vs baseline: 16.4496x; 1.1926x over previous
import functools

import jax
import jax.numpy as jnp
from jax import lax
from jax.experimental import pallas as pl
from jax.experimental.pallas import tpu as pltpu
from jax.experimental.pallas import tpu_sc as plsc

N_FINE = 16384
N_COARSE = 4096
D = 256
K = 3
BIG = 1e10
MASKED = 1e30

NC = 2
NS = 16
NW = NC * NS
L = 16

RF = 256
NBLK = N_FINE // RF
YBLK = N_COARSE // RF

CPW = N_COARSE // NW
CB = 32
PTS_W = N_FINE // NW
NCHUNK = PTS_W // CB

@functools.cache
def _sc_kernels():
    mesh = plsc.VectorSubcoreMesh(
        core_axis_name="c", subcore_axis_name="s",
        num_cores=NC, num_subcores=NS)

    @functools.partial(
        pl.kernel,
        out_type=[jax.ShapeDtypeStruct((N_COARSE,), jnp.float32)] * 4,
        mesh=mesh,
        scratch_types=[
            pltpu.VMEM((N_FINE,), jnp.float32),
            pltpu.VMEM((N_FINE,), jnp.float32),
            pltpu.VMEM((N_FINE,), jnp.float32),
            pltpu.VMEM((N_FINE,), jnp.float32),
            pltpu.VMEM((CPW,), jnp.int32),
            pltpu.VMEM((CPW,), jnp.float32),
            pltpu.VMEM((CPW,), jnp.float32),
            pltpu.VMEM((CPW,), jnp.float32),
            pltpu.VMEM((CPW,), jnp.float32),
        ],
        compiler_params=pltpu.CompilerParams(needs_layout_passes=False),
    )
    def _sc_gather_coarse(px, py, pz, pb, idxh, opx, opy, opz, opb,
                          tx, ty, tz, tb, idx_v, ox, oy, oz, ob):
        wid = lax.axis_index("s") * NC + lax.axis_index("c")
        base = wid * CPW
        pltpu.sync_copy(px, tx)
        pltpu.sync_copy(py, ty)
        pltpu.sync_copy(pz, tz)
        pltpu.sync_copy(pb, tb)
        pltpu.sync_copy(idxh.at[pl.ds(base, CPW)], idx_v)

        def body(j, _):
            sl = pl.ds(j * L, L)
            iv = idx_v[sl]
            ox[sl] = plsc.load_gather(tx, [iv])
            oy[sl] = plsc.load_gather(ty, [iv])
            oz[sl] = plsc.load_gather(tz, [iv])
            ob[sl] = plsc.load_gather(tb, [iv])
            return 0

        lax.fori_loop(0, CPW // L, body, 0)
        pltpu.sync_copy(ox, opx.at[pl.ds(base, CPW)])
        pltpu.sync_copy(oy, opy.at[pl.ds(base, CPW)])
        pltpu.sync_copy(oz, opz.at[pl.ds(base, CPW)])
        pltpu.sync_copy(ob, opb.at[pl.ds(base, CPW)])

    @functools.partial(
        pl.kernel,
        out_type=jax.ShapeDtypeStruct((N_FINE, D), jnp.float32),
        mesh=mesh,
        scratch_types=[
            pltpu.VMEM((3 * PTS_W,), jnp.int32),
            pltpu.VMEM((3 * PTS_W,), jnp.float32),
            pltpu.VMEM((3 * CB, D), jnp.float32),
            pltpu.VMEM((3 * CB, D), jnp.float32),
            pltpu.VMEM((CB, D), jnp.float32),
            pltpu.VMEM((CB, D), jnp.float32),
            pltpu.VMEM((CB, D), jnp.float32),
            pltpu.VMEM((CB, D), jnp.float32),
            pltpu.SemaphoreType.DMA,
            pltpu.SemaphoreType.DMA,
            pltpu.SemaphoreType.DMA,
            pltpu.SemaphoreType.DMA,
            pltpu.SemaphoreType.DMA,
            pltpu.SemaphoreType.DMA,
        ],
        compiler_params=pltpu.CompilerParams(needs_layout_passes=False),
    )
    def _sc_combine(y_hbm, z_hbm, nn_hbm, w_hbm, out_hbm,
                    idx_all, w_all, g0, g1, z0, z1, o0, o1,
                    gs0, gs1, zs0, zs1, ws0, ws1):
        wid = lax.axis_index("s") * NC + lax.axis_index("c")
        base = wid * PTS_W
        pltpu.sync_copy(nn_hbm.at[pl.ds(base * 3, 3 * PTS_W)], idx_all)
        pltpu.sync_copy(w_hbm.at[pl.ds(base * 3, 3 * PTS_W)], w_all)

        def issue(c, g_v, z_v, gsem, zsem):
            pltpu.async_copy(
                y_hbm.at[idx_all.at[pl.ds(c * (3 * CB), 3 * CB)]], g_v, gsem)
            pltpu.async_copy(z_hbm.at[pl.ds(base + c * CB, CB)], z_v, zsem)

        def wait_into(c, g_v, z_v, gsem, zsem):
            pltpu.make_async_copy(
                y_hbm.at[idx_all.at[pl.ds(c * (3 * CB), 3 * CB)]], g_v,
                gsem).wait()
            pltpu.make_async_copy(
                z_hbm.at[pl.ds(base + c * CB, CB)], z_v, zsem).wait()

        def drain_write(o_v, wsem):
            pltpu.make_async_copy(z_hbm.at[pl.ds(base, CB)], o_v, wsem).wait()

        def compute(c, g_v, z_v, o_v):
            def point_body(i, _):
                j0 = c * (3 * CB) + 3 * i
                w0 = plsc.load_gather(w_all, [jnp.full((L,), j0, jnp.int32)])
                w1 = plsc.load_gather(
                    w_all, [jnp.full((L,), j0 + 1, jnp.int32)])
                w2 = plsc.load_gather(
                    w_all, [jnp.full((L,), j0 + 2, jnp.int32)])
                j = 3 * i
                for v in range(D // L):
                    sl = pl.ds(v * L, L)
                    o_v[i, sl] = (z_v[i, sl] + w0 * g_v[j, sl]
                                  + w1 * g_v[j + 1, sl] + w2 * g_v[j + 2, sl])
                return 0

            lax.fori_loop(0, CB, point_body, 0)

        issue(0, g0, z0, gs0, zs0)

        def pair_body(c2, _):
            c = 2 * c2
            issue(c + 1, g1, z1, gs1, zs1)
            wait_into(c, g0, z0, gs0, zs0)

            @pl.when(c2 >= 1)
            def _():
                drain_write(o0, ws0)

            compute(c, g0, z0, o0)
            pltpu.async_copy(o0, out_hbm.at[pl.ds(base + c * CB, CB)], ws0)

            @pl.when(c2 < NCHUNK // 2 - 1)
            def _():
                issue(c + 2, g0, z0, gs0, zs0)

            wait_into(c + 1, g1, z1, gs1, zs1)

            @pl.when(c2 >= 1)
            def _():
                drain_write(o1, ws1)

            compute(c + 1, g1, z1, o1)
            pltpu.async_copy(o1, out_hbm.at[pl.ds(base + (c + 1) * CB, CB)],
                             ws1)
            return 0

        lax.fori_loop(0, NCHUNK // 2, pair_body, 0)
        drain_write(o0, ws0)
        drain_write(o1, ws1)

    return _sc_gather_coarse, _sc_combine


CW = 512
NCH = N_COARSE // CW


def _insert(m1, m2, m3, a1, a2, a3, v, iv):
    c1 = v < m1
    c2 = v < m2
    c3 = v < m3
    nm1 = jnp.where(c1, v, m1)
    na1 = jnp.where(c1, iv, a1)
    nm2 = jnp.where(c1, m1, jnp.where(c2, v, m2))
    na2 = jnp.where(c1, a1, jnp.where(c2, iv, a2))
    nm3 = jnp.where(c2, m2, jnp.where(c3, v, m3))
    na3 = jnp.where(c2, a2, jnp.where(c3, iv, a3))
    return nm1, nm2, nm3, na1, na2, na3


def _knn_mlp_body(posf_ref, batf_ref, posct3_ref, xs_ref, x_ref,
                  w_ref, b_ref, z_ref, y_ref, wk_ref, ak_ref):
    i = pl.program_id(0)
    pf = posf_ref[...]
    bf = batf_ref[...]
    b_min = batf_ref[0, 0]
    b_max = batf_ref[RF - 1, 0]
    bc_all = posct3_ref[3, :, :]
    lt = (bc_all < b_min).astype(jnp.int32)
    le = (bc_all <= b_max).astype(jnp.int32)
    c_lo = jnp.sum(jnp.sum(lt, axis=1, keepdims=True), axis=0,
                   keepdims=True)[0, 0]
    c_hi = jnp.sum(jnp.sum(le, axis=1, keepdims=True), axis=0,
                   keepdims=True)[0, 0]
    ch0 = c_lo // CW
    nch = (c_hi - ch0 * CW + (CW - 1)) // CW

    fiota_l = lax.broadcasted_iota(jnp.int32, (RF, CW), 1).astype(jnp.float32)

    def chunk_body(t, carry):
        m1, m2, m3, a1, a2, a3 = carry
        ch = ch0 + t
        d2 = None
        for d in range(3):
            pcd = posct3_ref[d, pl.ds(ch, 1), :]
            diff = pf[:, d:d + 1] - pcd
            d2 = diff * diff if d2 is None else d2 + diff * diff
        bc = posct3_ref[3, pl.ds(ch, 1), :]
        d2 = jnp.where(bf != bc, jnp.float32(BIG), d2)
        fiota = fiota_l + (ch * CW).astype(jnp.float32)
        cur = d2
        for k in range(K):
            m = jnp.min(cur, axis=1, keepdims=True)
            a = jnp.min(jnp.where(cur == m, fiota, jnp.float32(2 * N_COARSE)),
                        axis=1, keepdims=True)
            if k < K - 1:
                cur = jnp.where(fiota == a, jnp.float32(MASKED), cur)
            m1, m2, m3, a1, a2, a3 = _insert(m1, m2, m3, a1, a2, a3, m, a)
        return m1, m2, m3, a1, a2, a3

    init = (jnp.full((RF, 1), MASKED, jnp.float32),
            jnp.full((RF, 1), MASKED, jnp.float32),
            jnp.full((RF, 1), MASKED, jnp.float32),
            jnp.zeros((RF, 1), jnp.float32),
            jnp.ones((RF, 1), jnp.float32),
            jnp.full((RF, 1), 2.0, jnp.float32))
    m1, m2, m3, a1, a2, a3 = lax.fori_loop(0, nch, chunk_body, init)

    ms = [jnp.minimum(m, jnp.float32(BIG)) for m in (m1, m2, m3)]
    r = [1.0 / (m + jnp.float32(1e-8)) for m in ms]
    s = (r[0] + r[1]) + r[2] + jnp.float32(1e-16)
    wk_ref[...] = jnp.concatenate([ri / s for ri in r], axis=1)
    ak_ref[...] = jnp.concatenate(
        [a.astype(jnp.int32) for a in (a1, a2, a3)], axis=1)

    z_ref[...] = (jnp.dot(xs_ref[...], w_ref[D:, :],
                          preferred_element_type=jnp.float32) + b_ref[...])

    @pl.when(i < YBLK)
    def _():
        y_ref[...] = jnp.dot(x_ref[...], w_ref[:D, :],
                             preferred_element_type=jnp.float32)


def _tc_knn_mlp(pos_f, bat_f2, posct3, x_skip, x, W, b2):
    return pl.pallas_call(
        _knn_mlp_body,
        grid=(NBLK,),
        in_specs=[
            pl.BlockSpec((RF, 3), lambda i: (i, 0)),
            pl.BlockSpec((RF, 1), lambda i: (i, 0)),
            pl.BlockSpec((4, NCH, CW), lambda i: (0, 0, 0)),
            pl.BlockSpec((RF, D), lambda i: (i, 0)),
            pl.BlockSpec((RF, D), lambda i: (jnp.minimum(i, YBLK - 1), 0)),
            pl.BlockSpec((2 * D, D), lambda i: (0, 0)),
            pl.BlockSpec((1, D), lambda i: (0, 0)),
        ],
        out_specs=[
            pl.BlockSpec((RF, D), lambda i: (i, 0)),
            pl.BlockSpec((RF, D), lambda i: (jnp.minimum(i, YBLK - 1), 0)),
            pl.BlockSpec((RF, K), lambda i: (i, 0)),
            pl.BlockSpec((RF, K), lambda i: (i, 0)),
        ],
        out_shape=[
            jax.ShapeDtypeStruct((N_FINE, D), jnp.float32),
            jax.ShapeDtypeStruct((N_COARSE, D), jnp.float32),
            jax.ShapeDtypeStruct((N_FINE, K), jnp.float32),
            jax.ShapeDtypeStruct((N_FINE, K), jnp.int32),
        ],
        compiler_params=pltpu.CompilerParams(
            dimension_semantics=("arbitrary",)),
    )(pos_f, bat_f2, posct3, x_skip, x, W, b2)


def kernel(x, idx, x_skip, pos_skip, batch_skip, point2curveidx_skip, W, b):
    _sc_gather_coarse, _sc_combine = _sc_kernels()
    idx32 = idx.astype(jnp.int32)
    batf = batch_skip.astype(jnp.float32)
    px = pos_skip[:, 0]
    py = pos_skip[:, 1]
    pz = pos_skip[:, 2]

    pcx, pcy, pcz, pcb = _sc_gather_coarse(px, py, pz, batf, idx32)
    posct3 = jnp.stack([pcx, pcy, pcz, pcb], axis=0).reshape(4, NCH, CW)

    z, y, wk, ak = _tc_knn_mlp(pos_skip, batf.reshape(N_FINE, 1), posct3,
                               x_skip, x, W, b.reshape(1, D))

    out = _sc_combine(y, z, ak.reshape(-1), wk.reshape(-1))
    return (out, pos_skip, batch_skip, point2curveidx_skip)


# speedup vs baseline: 18.5810x; 1.1296x over previous
import functools

import jax
import jax.numpy as jnp
from jax import lax
from jax.experimental import pallas as pl
from jax.experimental.pallas import tpu as pltpu
from jax.experimental.pallas import tpu_sc as plsc

N_FINE = 16384
N_COARSE = 4096
D = 256
K = 3
BIG = 1e10
MASKED = 1e30

NC = 2
NS = 16
NW = NC * NS
L = 16

RF = 512
NBLK = N_FINE // RF
YBLK = N_COARSE // RF

CPW = N_COARSE // NW
CB = 32
PTS_W = N_FINE // NW
NCHUNK = PTS_W // CB

@functools.cache
def _sc_kernels():
    mesh = plsc.VectorSubcoreMesh(
        core_axis_name="c", subcore_axis_name="s",
        num_cores=NC, num_subcores=NS)

    @functools.partial(
        pl.kernel,
        out_type=[jax.ShapeDtypeStruct((N_COARSE,), jnp.float32)] * 4,
        mesh=mesh,
        scratch_types=[
            pltpu.VMEM((N_FINE,), jnp.float32),
            pltpu.VMEM((N_FINE,), jnp.float32),
            pltpu.VMEM((N_FINE,), jnp.float32),
            pltpu.VMEM((N_FINE,), jnp.float32),
            pltpu.VMEM((CPW,), jnp.int32),
            pltpu.VMEM((CPW,), jnp.float32),
            pltpu.VMEM((CPW,), jnp.float32),
            pltpu.VMEM((CPW,), jnp.float32),
            pltpu.VMEM((CPW,), jnp.float32),
        ],
        compiler_params=pltpu.CompilerParams(needs_layout_passes=False),
    )
    def _sc_gather_coarse(px, py, pz, pb, idxh, opx, opy, opz, opb,
                          tx, ty, tz, tb, idx_v, ox, oy, oz, ob):
        wid = lax.axis_index("s") * NC + lax.axis_index("c")
        base = wid * CPW
        pltpu.sync_copy(px, tx)
        pltpu.sync_copy(py, ty)
        pltpu.sync_copy(pz, tz)
        pltpu.sync_copy(pb, tb)
        pltpu.sync_copy(idxh.at[pl.ds(base, CPW)], idx_v)

        def body(j, _):
            sl = pl.ds(j * L, L)
            iv = idx_v[sl]
            ox[sl] = plsc.load_gather(tx, [iv])
            oy[sl] = plsc.load_gather(ty, [iv])
            oz[sl] = plsc.load_gather(tz, [iv])
            ob[sl] = plsc.load_gather(tb, [iv])
            return 0

        lax.fori_loop(0, CPW // L, body, 0)
        pltpu.sync_copy(ox, opx.at[pl.ds(base, CPW)])
        pltpu.sync_copy(oy, opy.at[pl.ds(base, CPW)])
        pltpu.sync_copy(oz, opz.at[pl.ds(base, CPW)])
        pltpu.sync_copy(ob, opb.at[pl.ds(base, CPW)])

    @functools.partial(
        pl.kernel,
        out_type=jax.ShapeDtypeStruct((N_FINE, D), jnp.float32),
        mesh=mesh,
        scratch_types=[
            pltpu.VMEM((3 * PTS_W,), jnp.int32),
            pltpu.VMEM((3 * PTS_W,), jnp.float32),
            pltpu.VMEM((3 * CB, D), jnp.float32),
            pltpu.VMEM((3 * CB, D), jnp.float32),
            pltpu.VMEM((CB, D), jnp.float32),
            pltpu.VMEM((CB, D), jnp.float32),
            pltpu.VMEM((CB, D), jnp.float32),
            pltpu.VMEM((CB, D), jnp.float32),
            pltpu.SemaphoreType.DMA,
            pltpu.SemaphoreType.DMA,
            pltpu.SemaphoreType.DMA,
            pltpu.SemaphoreType.DMA,
            pltpu.SemaphoreType.DMA,
            pltpu.SemaphoreType.DMA,
        ],
        compiler_params=pltpu.CompilerParams(needs_layout_passes=False),
    )
    def _sc_combine(y_hbm, z_hbm, nn_hbm, w_hbm, out_hbm,
                    idx_all, w_all, g0, g1, z0, z1, o0, o1,
                    gs0, gs1, zs0, zs1, ws0, ws1):
        wid = lax.axis_index("s") * NC + lax.axis_index("c")
        base = wid * PTS_W
        pltpu.sync_copy(nn_hbm.at[pl.ds(base * 3, 3 * PTS_W)], idx_all)
        pltpu.sync_copy(w_hbm.at[pl.ds(base * 3, 3 * PTS_W)], w_all)

        def issue(c, g_v, z_v, gsem, zsem):
            pltpu.async_copy(
                y_hbm.at[idx_all.at[pl.ds(c * (3 * CB), 3 * CB)]], g_v, gsem)
            pltpu.async_copy(z_hbm.at[pl.ds(base + c * CB, CB)], z_v, zsem)

        def wait_into(c, g_v, z_v, gsem, zsem):
            pltpu.make_async_copy(
                y_hbm.at[idx_all.at[pl.ds(c * (3 * CB), 3 * CB)]], g_v,
                gsem).wait()
            pltpu.make_async_copy(
                z_hbm.at[pl.ds(base + c * CB, CB)], z_v, zsem).wait()

        def drain_write(o_v, wsem):
            pltpu.make_async_copy(z_hbm.at[pl.ds(base, CB)], o_v, wsem).wait()

        def compute(c, g_v, z_v, o_v):
            def point_body(i, _):
                j0 = c * (3 * CB) + 3 * i
                w0 = plsc.load_gather(w_all, [jnp.full((L,), j0, jnp.int32)])
                w1 = plsc.load_gather(
                    w_all, [jnp.full((L,), j0 + 1, jnp.int32)])
                w2 = plsc.load_gather(
                    w_all, [jnp.full((L,), j0 + 2, jnp.int32)])
                j = 3 * i
                for v in range(D // L):
                    sl = pl.ds(v * L, L)
                    o_v[i, sl] = (z_v[i, sl] + w0 * g_v[j, sl]
                                  + w1 * g_v[j + 1, sl] + w2 * g_v[j + 2, sl])
                return 0

            lax.fori_loop(0, CB, point_body, 0)

        issue(0, g0, z0, gs0, zs0)

        def pair_body(c2, _):
            c = 2 * c2
            issue(c + 1, g1, z1, gs1, zs1)
            wait_into(c, g0, z0, gs0, zs0)

            @pl.when(c2 >= 1)
            def _():
                drain_write(o0, ws0)

            compute(c, g0, z0, o0)
            pltpu.async_copy(o0, out_hbm.at[pl.ds(base + c * CB, CB)], ws0)

            @pl.when(c2 < NCHUNK // 2 - 1)
            def _():
                issue(c + 2, g0, z0, gs0, zs0)

            wait_into(c + 1, g1, z1, gs1, zs1)

            @pl.when(c2 >= 1)
            def _():
                drain_write(o1, ws1)

            compute(c + 1, g1, z1, o1)
            pltpu.async_copy(o1, out_hbm.at[pl.ds(base + (c + 1) * CB, CB)],
                             ws1)
            return 0

        lax.fori_loop(0, NCHUNK // 2, pair_body, 0)
        drain_write(o0, ws0)
        drain_write(o1, ws1)

    return _sc_gather_coarse, _sc_combine


CW = 512
NCH = N_COARSE // CW


def _insert(m1, m2, m3, a1, a2, a3, v, iv):
    c1 = v < m1
    c2 = v < m2
    c3 = v < m3
    nm1 = jnp.where(c1, v, m1)
    na1 = jnp.where(c1, iv, a1)
    nm2 = jnp.where(c1, m1, jnp.where(c2, v, m2))
    na2 = jnp.where(c1, a1, jnp.where(c2, iv, a2))
    nm3 = jnp.where(c2, m2, jnp.where(c3, v, m3))
    na3 = jnp.where(c2, a2, jnp.where(c3, iv, a3))
    return nm1, nm2, nm3, na1, na2, na3


def _knn_mlp_body(posf_ref, batf_ref, posct3_ref, xs_ref, x_ref,
                  w_ref, b_ref, z_ref, y_ref, wk_ref, ak_ref):
    i = pl.program_id(0)
    pf = posf_ref[...]
    bf = batf_ref[...]
    b_min = batf_ref[0, 0]
    b_max = batf_ref[RF - 1, 0]
    bc_all = posct3_ref[3, :, :]
    lt = (bc_all < b_min).astype(jnp.int32)
    le = (bc_all <= b_max).astype(jnp.int32)
    c_lo = jnp.sum(jnp.sum(lt, axis=1, keepdims=True), axis=0,
                   keepdims=True)[0, 0]
    c_hi = jnp.sum(jnp.sum(le, axis=1, keepdims=True), axis=0,
                   keepdims=True)[0, 0]
    ch0 = c_lo // CW
    nch = (c_hi - ch0 * CW + (CW - 1)) // CW

    fiota_l = lax.broadcasted_iota(jnp.int32, (RF, CW), 1).astype(jnp.float32)
    pfm2 = pf * jnp.float32(-2.0)
    pf2 = jnp.sum(pf * pf, axis=1, keepdims=True)

    def chunk_body(t, carry):
        m1, m2, m3, a1, a2, a3 = carry
        ch = ch0 + t
        pc3 = posct3_ref[0:3, pl.ds(ch, 1), :].reshape(3, CW)
        pc2 = jnp.sum(pc3 * pc3, axis=0, keepdims=True)
        cross = jnp.dot(pfm2, pc3, preferred_element_type=jnp.float32)
        d2 = (cross + pf2) + pc2
        bc = posct3_ref[3, pl.ds(ch, 1), :]
        d2 = jnp.where(bf != bc, jnp.float32(BIG), d2)
        chf = (ch * CW).astype(jnp.float32)
        cur = d2
        for k in range(K):
            m = jnp.min(cur, axis=1, keepdims=True)
            a = jnp.min(jnp.where(cur == m, fiota_l, jnp.float32(2 * N_COARSE)),
                        axis=1, keepdims=True)
            if k < K - 1:
                cur = jnp.where(fiota_l == a, jnp.float32(MASKED), cur)
            m1, m2, m3, a1, a2, a3 = _insert(m1, m2, m3, a1, a2, a3, m,
                                             a + chf)
        return m1, m2, m3, a1, a2, a3

    init = (jnp.full((RF, 1), MASKED, jnp.float32),
            jnp.full((RF, 1), MASKED, jnp.float32),
            jnp.full((RF, 1), MASKED, jnp.float32),
            jnp.zeros((RF, 1), jnp.float32),
            jnp.ones((RF, 1), jnp.float32),
            jnp.full((RF, 1), 2.0, jnp.float32))
    m1, m2, m3, a1, a2, a3 = lax.fori_loop(0, nch, chunk_body, init)

    ms = [jnp.minimum(m, jnp.float32(BIG)) for m in (m1, m2, m3)]
    r = [1.0 / (m + jnp.float32(1e-8)) for m in ms]
    s = (r[0] + r[1]) + r[2] + jnp.float32(1e-16)
    wk_ref[...] = jnp.concatenate([ri / s for ri in r], axis=1)
    ak_ref[...] = jnp.concatenate(
        [a.astype(jnp.int32) for a in (a1, a2, a3)], axis=1)

    z_ref[...] = (jnp.dot(xs_ref[...], w_ref[D:, :],
                          preferred_element_type=jnp.float32) + b_ref[...])

    @pl.when(i < YBLK)
    def _():
        y_ref[...] = jnp.dot(x_ref[...], w_ref[:D, :],
                             preferred_element_type=jnp.float32)


def _tc_knn_mlp(pos_f, bat_f2, posct3, x_skip, x, W, b2):
    return pl.pallas_call(
        _knn_mlp_body,
        grid=(NBLK,),
        in_specs=[
            pl.BlockSpec((RF, 3), lambda i: (i, 0)),
            pl.BlockSpec((RF, 1), lambda i: (i, 0)),
            pl.BlockSpec((4, NCH, CW), lambda i: (0, 0, 0)),
            pl.BlockSpec((RF, D), lambda i: (i, 0)),
            pl.BlockSpec((RF, D), lambda i: (jnp.minimum(i, YBLK - 1), 0)),
            pl.BlockSpec((2 * D, D), lambda i: (0, 0)),
            pl.BlockSpec((1, D), lambda i: (0, 0)),
        ],
        out_specs=[
            pl.BlockSpec((RF, D), lambda i: (i, 0)),
            pl.BlockSpec((RF, D), lambda i: (jnp.minimum(i, YBLK - 1), 0)),
            pl.BlockSpec((RF, K), lambda i: (i, 0)),
            pl.BlockSpec((RF, K), lambda i: (i, 0)),
        ],
        out_shape=[
            jax.ShapeDtypeStruct((N_FINE, D), jnp.float32),
            jax.ShapeDtypeStruct((N_COARSE, D), jnp.float32),
            jax.ShapeDtypeStruct((N_FINE, K), jnp.float32),
            jax.ShapeDtypeStruct((N_FINE, K), jnp.int32),
        ],
        compiler_params=pltpu.CompilerParams(
            dimension_semantics=("arbitrary",)),
    )(pos_f, bat_f2, posct3, x_skip, x, W, b2)


def kernel(x, idx, x_skip, pos_skip, batch_skip, point2curveidx_skip, W, b):
    _sc_gather_coarse, _sc_combine = _sc_kernels()
    idx32 = idx.astype(jnp.int32)
    batf = batch_skip.astype(jnp.float32)
    px = pos_skip[:, 0]
    py = pos_skip[:, 1]
    pz = pos_skip[:, 2]

    pcx, pcy, pcz, pcb = _sc_gather_coarse(px, py, pz, batf, idx32)
    posct3 = jnp.stack([pcx, pcy, pcz, pcb], axis=0).reshape(4, NCH, CW)

    z, y, wk, ak = _tc_knn_mlp(pos_skip, batf.reshape(N_FINE, 1), posct3,
                               x_skip, x, W, b.reshape(1, D))

    out = _sc_combine(y, z, ak.reshape(-1), wk.reshape(-1))
    return (out, pos_skip, batch_skip, point2curveidx_skip)
